# baseline (speedup 1.0000x reference)
import functools

import jax
import jax.numpy as jnp
from jax import lax
from jax.experimental import pallas as pl
from jax.experimental.pallas import tpu as pltpu
from jax.experimental.pallas import tpu_sc as plsc

NUM_CORES = 2
NUM_SUBCORES = 16
NUM_WORKERS = NUM_CORES * NUM_SUBCORES
LANES = 16

IDX_MINOR = 128


def _rsqrt16(x):
    xi = plsc.bitcast(x, jnp.int32)
    yi = jnp.int32(0x5F3759DF) - (xi >> 1)
    y = plsc.bitcast(yi, jnp.float32)
    for _ in range(3):
        y = y * (1.5 - 0.5 * x * y * y)
    return y


def _make_sc_embed(batch, hist, dim):
    nb = batch // NUM_WORKERS
    slab = nb // IDX_MINOR
    CB = 256
    n_streams = CB // IDX_MINOR
    groups = CB // LANES
    halves = nb // CB
    n_chunks = hist * halves

    mesh = plsc.VectorSubcoreMesh(core_axis_name="c", subcore_axis_name="s")

    @functools.partial(
        pl.kernel,
        out_type=jax.ShapeDtypeStruct((hist, dim, batch), jnp.float32),
        mesh=mesh,
        scratch_types=[
            pltpu.VMEM((hist, slab, IDX_MINOR), jnp.int32),
            pltpu.VMEM((2, CB, dim), jnp.float32),
            pltpu.VMEM((2, 1, dim, CB), jnp.float32),
            pltpu.SemaphoreType.DMA,
            pltpu.SemaphoreType.DMA,
            pltpu.SemaphoreType.DMA,
            pltpu.SemaphoreType.DMA,
        ],
        compiler_params=pltpu.CompilerParams(
            needs_layout_passes=False, use_tc_tiling_on_sc=False
        ),
    )
    def sc_embed(idx_hbm, table_hbm, out_hbm, idx_v, rows_v, outt_v,
                 gsem0, gsem1, osem0, osem1):
        wid = lax.axis_index("s") * NUM_CORES + lax.axis_index("c")
        lane = jnp.arange(LANES, dtype=jnp.int32)
        zero16 = jnp.zeros((LANES,), jnp.int32)
        b0 = wid * nb
        gsems = (gsem0, gsem1)
        osems = (osem0, osem1)

        pltpu.sync_copy(idx_hbm.at[:, pl.ds(wid * slab, slab)], idx_v)

        def fire_gather(c, buf):
            h = c // halves
            half = c % halves
            for j in range(n_streams):
                pltpu.async_copy(
                    table_hbm.at[idx_v.at[h, half * n_streams + j]],
                    rows_v.at[buf].at[pl.ds(j * IDX_MINOR, IDX_MINOR)],
                    gsems[buf],
                )

        def wait_gather(c, buf):
            h = c // halves
            half = c % halves
            for j in range(n_streams):
                pltpu.make_async_copy(
                    table_hbm.at[idx_v.at[h, half * n_streams + j]],
                    rows_v.at[buf].at[pl.ds(j * IDX_MINOR, IDX_MINOR)],
                    gsems[buf],
                ).wait()

        def out_slice(c):
            h = c // halves
            half = c % halves
            return out_hbm.at[pl.ds(h, 1), :, pl.ds(b0 + half * CB, CB)]

        def compute(buf):
            rbuf = rows_v.at[buf]
            obuf = outt_v.at[buf]

            def do_group(g, c2):
                rids = g * LANES + lane
                accs = [jnp.zeros((LANES,), jnp.float32) for _ in range(4)]
                for c in range(dim):
                    col = (lane + c) & (dim - 1)
                    v = plsc.load_gather(rbuf, [rids, col])
                    accs[c % 4] = accs[c % 4] + v * v
                acc = (accs[0] + accs[1]) + (accs[2] + accs[3])
                rinv = jnp.minimum(_rsqrt16(acc), jnp.float32(1e12))
                for c in range(dim):
                    col = (lane + c) & (dim - 1)
                    v = plsc.load_gather(rbuf, [rids, col])
                    plsc.store_scatter(obuf, [zero16, col, rids], v * rinv)
                return c2

            lax.fori_loop(0, groups, do_group, 0)

        def half_step(c, buf, tt):
            @pl.when(tt > 0)
            def _():
                pltpu.make_async_copy(
                    outt_v.at[buf], out_slice(c), osems[buf]
                ).wait()

            compute(buf)
            pltpu.async_copy(outt_v.at[buf], out_slice(c), osems[buf])

        fire_gather(0, 0)

        def body(tt, carry):
            c0 = 2 * tt
            c1 = c0 + 1
            fire_gather(c1, 1)
            wait_gather(c0, 0)
            half_step(c0, 0, tt)

            @pl.when(tt < n_chunks // 2 - 1)
            def _():
                fire_gather(c0 + 2, 0)

            wait_gather(c1, 1)
            half_step(c1, 1, tt)
            return carry

        lax.fori_loop(0, n_chunks // 2, body, 0)
        pltpu.make_async_copy(outt_v.at[0], out_slice(n_chunks - 2), osem0).wait()
        pltpu.make_async_copy(outt_v.at[1], out_slice(n_chunks - 1), osem1).wait()

    return sc_embed


def kernel(input_ids, table):
    batch, hist = input_ids.shape
    vocab, dim = table.shape
    idx_t = (
        input_ids.astype(jnp.int32)
        .T.reshape(hist, batch // IDX_MINOR, IDX_MINOR)
    )
    out_t = _make_sc_embed(batch, hist, dim)(idx_t, table)
    return out_t.transpose(2, 0, 1)


# speedup vs baseline: 1.1836x; 1.1836x over previous
import functools

import jax
import jax.numpy as jnp
from jax import lax
from jax.experimental import pallas as pl
from jax.experimental.pallas import tpu as pltpu
from jax.experimental.pallas import tpu_sc as plsc

NUM_CORES = 2
NUM_SUBCORES = 16
NUM_WORKERS = NUM_CORES * NUM_SUBCORES
LANES = 16

IDX_MINOR = 128


def _rsqrt16(x):
    xi = plsc.bitcast(x, jnp.int32)
    yi = jnp.int32(0x5F3759DF) - (xi >> 1)
    y = plsc.bitcast(yi, jnp.float32)
    for _ in range(3):
        y = y * (1.5 - 0.5 * x * y * y)
    return y


def _make_sc_embed(batch, hist, dim):
    nb = batch // NUM_WORKERS
    slab = nb // IDX_MINOR
    CB = 256
    n_streams = CB // IDX_MINOR
    groups = CB // LANES
    halves = nb // CB
    n_chunks = hist * halves

    mesh = plsc.VectorSubcoreMesh(core_axis_name="c", subcore_axis_name="s")

    @functools.partial(
        pl.kernel,
        out_type=jax.ShapeDtypeStruct((hist, dim, batch), jnp.float32),
        mesh=mesh,
        scratch_types=[
            pltpu.VMEM((hist, slab, IDX_MINOR), jnp.int32),
            pltpu.VMEM((2, CB, dim), jnp.float32),
            pltpu.VMEM((2, 1, dim, CB), jnp.float32),
            pltpu.VMEM((40, LANES), jnp.int32),
            pltpu.SemaphoreType.DMA,
            pltpu.SemaphoreType.DMA,
            pltpu.SemaphoreType.DMA,
            pltpu.SemaphoreType.DMA,
        ],
        compiler_params=pltpu.CompilerParams(
            needs_layout_passes=False,
            use_tc_tiling_on_sc=False,
            disable_bounds_checks=True,
        ),
    )
    def sc_embed(idx_hbm, table_hbm, out_hbm, idx_v, rows_v, outt_v, tabs,
                 gsem0, gsem1, osem0, osem1):
        wid = lax.axis_index("s") * NUM_CORES + lax.axis_index("c")
        lane = jnp.arange(LANES, dtype=jnp.int32)
        zero16 = jnp.zeros((LANES,), jnp.int32)
        b0 = wid * nb
        gsems = (gsem0, gsem1)
        osems = (osem0, osem1)

        pltpu.sync_copy(idx_hbm.at[:, pl.ds(wid * slab, slab)], idx_v)

        for k in range(LANES):
            tabs[k, :] = lane * dim + (lane ^ k)
            tabs[LANES + k, :] = (lane ^ k) * CB + lane
        for j in range(4):
            tabs[32 + j, :] = jnp.full((LANES,), 16 * j, jnp.int32)
            tabs[36 + j, :] = jnp.full((LANES,), 16 * j * CB, jnp.int32)

        def fire_gather(c, buf):
            h = c // halves
            half = c % halves
            for j in range(n_streams):
                pltpu.async_copy(
                    table_hbm.at[idx_v.at[h, half * n_streams + j]],
                    rows_v.at[buf].at[pl.ds(j * IDX_MINOR, IDX_MINOR)],
                    gsems[buf],
                )

        def wait_gather(c, buf):
            h = c // halves
            half = c % halves
            for j in range(n_streams):
                pltpu.make_async_copy(
                    table_hbm.at[idx_v.at[h, half * n_streams + j]],
                    rows_v.at[buf].at[pl.ds(j * IDX_MINOR, IDX_MINOR)],
                    gsems[buf],
                ).wait()

        def out_slice(c):
            h = c // halves
            half = c % halves
            return out_hbm.at[pl.ds(h, 1), :, pl.ds(b0 + half * CB, CB)]

        def compute(buf):
            rbuf = rows_v.at[buf]
            obuf = outt_v.at[buf]

            def do_group(g, c2):
                bx = [tabs[k, :] + g * (LANES * dim) for k in range(LANES)]
                offs = [tabs[32 + j, :] for j in range(4)]
                acc = jnp.zeros((LANES,), jnp.float32)
                for c in range(dim):
                    a = bx[c & 15] + offs[c >> 4]
                    v = plsc.load_gather(rbuf, [zero16, a])
                    acc = acc + v * v
                rinv = jnp.minimum(_rsqrt16(acc), jnp.float32(1e12))
                by = [tabs[LANES + k, :] + g * LANES for k in range(LANES)]
                offos = [tabs[36 + j, :] for j in range(4)]
                for c in range(dim):
                    a = bx[c & 15] + offs[c >> 4]
                    v = plsc.load_gather(rbuf, [zero16, a])
                    ao = by[c & 15] + offos[c >> 4]
                    plsc.store_scatter(obuf, [zero16, zero16, ao], v * rinv)
                return c2

            lax.fori_loop(0, groups, do_group, 0)

        def half_step(c, buf, tt):
            @pl.when(tt > 0)
            def _():
                pltpu.make_async_copy(
                    outt_v.at[buf], out_slice(c), osems[buf]
                ).wait()

            compute(buf)
            pltpu.async_copy(outt_v.at[buf], out_slice(c), osems[buf])

        fire_gather(0, 0)

        def body(tt, carry):
            c0 = 2 * tt
            c1 = c0 + 1
            fire_gather(c1, 1)
            wait_gather(c0, 0)
            half_step(c0, 0, tt)

            @pl.when(tt < n_chunks // 2 - 1)
            def _():
                fire_gather(c0 + 2, 0)

            wait_gather(c1, 1)
            half_step(c1, 1, tt)
            return carry

        lax.fori_loop(0, n_chunks // 2, body, 0)
        pltpu.make_async_copy(outt_v.at[0], out_slice(n_chunks - 2), osem0).wait()
        pltpu.make_async_copy(outt_v.at[1], out_slice(n_chunks - 1), osem1).wait()

    return sc_embed


def kernel(input_ids, table):
    batch, hist = input_ids.shape
    vocab, dim = table.shape
    idx_t = (
        input_ids.astype(jnp.int32)
        .T.reshape(hist, batch // IDX_MINOR, IDX_MINOR)
    )
    out_t = _make_sc_embed(batch, hist, dim)(idx_t, table)
    return out_t.transpose(2, 0, 1)


# speedup vs baseline: 1.3887x; 1.1733x over previous
import functools

import jax
import jax.numpy as jnp
from jax import lax
from jax.experimental import pallas as pl
from jax.experimental.pallas import tpu as pltpu
from jax.experimental.pallas import tpu_sc as plsc

NUM_CORES = 2
NUM_SUBCORES = 16
NUM_WORKERS = NUM_CORES * NUM_SUBCORES
LANES = 16

IDX_MINOR = 128


def _rsqrt16(x):
    xi = plsc.bitcast(x, jnp.int32)
    yi = jnp.int32(0x5F3759DF) - (xi >> 1)
    y = plsc.bitcast(yi, jnp.float32)
    for _ in range(3):
        y = y * (1.5 - 0.5 * x * y * y)
    return y


def _make_sc_embed(batch, hist, dim):
    nb = batch // NUM_WORKERS
    slab = nb // IDX_MINOR
    CB = 256
    n_streams = CB // IDX_MINOR
    groups = CB // LANES
    halves = nb // CB
    n_chunks = hist * halves

    mesh = plsc.VectorSubcoreMesh(core_axis_name="c", subcore_axis_name="s")

    nbb = batch // 128
    cbb = CB // 128

    @functools.partial(
        pl.kernel,
        out_type=jax.ShapeDtypeStruct((hist, dim // 8, nbb, 8, 128), jnp.float32),
        mesh=mesh,
        scratch_types=[
            pltpu.VMEM((hist, slab, IDX_MINOR), jnp.int32),
            pltpu.VMEM((2, CB, dim), jnp.float32),
            pltpu.VMEM((2, 1, dim // 8, cbb, 8, 128), jnp.float32),
            pltpu.VMEM((40, LANES), jnp.int32),
            pltpu.SemaphoreType.DMA,
            pltpu.SemaphoreType.DMA,
            pltpu.SemaphoreType.DMA,
            pltpu.SemaphoreType.DMA,
        ],
        compiler_params=pltpu.CompilerParams(
            needs_layout_passes=False,
            use_tc_tiling_on_sc=False,
            disable_bounds_checks=True,
        ),
    )
    def sc_embed(idx_hbm, table_hbm, out_hbm, idx_v, rows_v, outt_v, tabs,
                 gsem0, gsem1, osem0, osem1):
        wid = lax.axis_index("s") * NUM_CORES + lax.axis_index("c")
        lane = jnp.arange(LANES, dtype=jnp.int32)
        zero16 = jnp.zeros((LANES,), jnp.int32)
        b0 = wid * nb
        gsems = (gsem0, gsem1)
        osems = (osem0, osem1)

        pltpu.sync_copy(idx_hbm.at[:, pl.ds(wid * slab, slab)], idx_v)

        for k in range(LANES):
            cl = lane ^ k
            tabs[k, :] = lane * dim + cl
            tabs[LANES + k, :] = (cl >> 3) * 2048 + (cl & 7) * 128 + lane
        for j in range(4):
            tabs[32 + j, :] = jnp.full((LANES,), 16 * j, jnp.int32)
            tabs[36 + j, :] = jnp.full((LANES,), j * 4096, jnp.int32)

        def fire_gather(c, buf):
            h = c // halves
            half = c % halves
            for j in range(n_streams):
                pltpu.async_copy(
                    table_hbm.at[idx_v.at[h, half * n_streams + j]],
                    rows_v.at[buf].at[pl.ds(j * IDX_MINOR, IDX_MINOR)],
                    gsems[buf],
                )

        def wait_gather(c, buf):
            h = c // halves
            half = c % halves
            for j in range(n_streams):
                pltpu.make_async_copy(
                    table_hbm.at[idx_v.at[h, half * n_streams + j]],
                    rows_v.at[buf].at[pl.ds(j * IDX_MINOR, IDX_MINOR)],
                    gsems[buf],
                ).wait()

        def out_slice(c):
            h = c // halves
            half = c % halves
            bb0 = wid * (nb // 128) + half * cbb
            return out_hbm.at[pl.ds(h, 1), :, pl.ds(bb0, cbb), :, :]

        def compute(buf):
            rbuf = rows_v.at[buf]
            obuf = outt_v.at[buf]

            def do_group(g, c2):
                bx = [tabs[k, :] + g * (LANES * dim) for k in range(LANES)]
                offs = [tabs[32 + j, :] for j in range(4)]
                acc = jnp.zeros((LANES,), jnp.float32)
                for c in range(dim):
                    a = bx[c & 15] + offs[c >> 4]
                    v = plsc.load_gather(rbuf, [zero16, a])
                    acc = acc + v * v
                rinv = jnp.minimum(_rsqrt16(acc), jnp.float32(1e12))
                goff = (g >> 3) * 1024 + (g & 7) * LANES
                by = [tabs[LANES + k, :] + goff for k in range(LANES)]
                offos = [tabs[36 + j, :] for j in range(4)]
                for c in range(dim):
                    a = bx[c & 15] + offs[c >> 4]
                    v = plsc.load_gather(rbuf, [zero16, a])
                    ao = by[c & 15] + offos[c >> 4]
                    plsc.store_scatter(
                        obuf, [zero16, zero16, zero16, zero16, ao], v * rinv
                    )
                return c2

            lax.fori_loop(0, groups, do_group, 0)

        def half_step(c, buf, tt):
            @pl.when(tt > 0)
            def _():
                pltpu.make_async_copy(
                    outt_v.at[buf], out_slice(c), osems[buf]
                ).wait()

            compute(buf)
            pltpu.async_copy(outt_v.at[buf], out_slice(c), osems[buf])

        fire_gather(0, 0)

        def body(tt, carry):
            c0 = 2 * tt
            c1 = c0 + 1
            fire_gather(c1, 1)
            wait_gather(c0, 0)
            half_step(c0, 0, tt)

            @pl.when(tt < n_chunks // 2 - 1)
            def _():
                fire_gather(c0 + 2, 0)

            wait_gather(c1, 1)
            half_step(c1, 1, tt)
            return carry

        lax.fori_loop(0, n_chunks // 2, body, 0)
        pltpu.make_async_copy(outt_v.at[0], out_slice(n_chunks - 2), osem0).wait()
        pltpu.make_async_copy(outt_v.at[1], out_slice(n_chunks - 1), osem1).wait()

    return sc_embed


def kernel(input_ids, table):
    batch, hist = input_ids.shape
    vocab, dim = table.shape
    idx_t = (
        input_ids.astype(jnp.int32)
        .T.reshape(hist, batch // IDX_MINOR, IDX_MINOR)
    )
    out5 = _make_sc_embed(batch, hist, dim)(idx_t, table)
    return out5.transpose(2, 4, 0, 1, 3).reshape(batch, hist, dim)
